# baseline (speedup 1.0000x reference)
import functools

import jax
import jax.numpy as jnp
from jax import lax
from jax.experimental import pallas as pl
from jax.experimental.pallas import tpu as pltpu
from jax.experimental.pallas import tpu_sc as plsc

B, NA, NB, D = 8, 1000, 1000, 256
NW = 32
PAD = 8192
ROWS_PER_W = PAD // NW
CHUNK = 128


def _tc_argmax_body(thr_ref, iou_ref, idx_ref, mask_ref):
    x = iou_ref[0]
    col = lax.broadcasted_iota(jnp.int32, (NA, NB), 1)
    mx = jnp.max(x, axis=1, keepdims=True)
    cand = jnp.where(x == mx, col, NB)
    jst = jnp.min(cand, axis=1, keepdims=True)
    b = pl.program_id(0)
    idx_ref[...] = (jst + b * NB).reshape(1, NA, 1)
    mask_ref[...] = (mx >= thr_ref[0]).astype(jnp.float32).reshape(1, NA, 1)


def _tc_cosine_body(ap_ref, az_ref, gp_ref, gz_ref, m_ref, sa_ref, sb_ref, c_ref):
    ap = ap_ref[0]
    az = az_ref[0]
    gp = gp_ref[...]
    gz = gz_ref[...]
    m = m_ref[0]

    def nrm(x):
        n = jnp.sqrt(jnp.sum(x * x, axis=1, keepdims=True))
        return x / jnp.maximum(n, 1e-12)

    ca = jnp.sum(nrm(ap) * nrm(gz), axis=1, keepdims=True)
    cb = jnp.sum(nrm(gp) * nrm(az), axis=1, keepdims=True)
    sa_ref[...] = jnp.broadcast_to(jnp.sum(m * ca), (1, 8, 128))
    sb_ref[...] = jnp.broadcast_to(jnp.sum(m * cb), (1, 8, 128))
    c_ref[...] = jnp.broadcast_to(jnp.sum(m), (1, 8, 128))


def _sc_gather_body(tp_hbm, tz_hbm, idx_hbm, gp_hbm, gz_hbm,
                    idx_v, rp_v, rz_v, s1, s2):
    wid = lax.axis_index("s") * 2 + lax.axis_index("c")
    pltpu.sync_copy(idx_hbm.at[pl.ds(wid * 2, 2)], idx_v)
    for h in range(ROWS_PER_W // CHUNK):
        cp1 = pltpu.async_copy(tp_hbm.at[idx_v.at[h]], rp_v, s1)
        cp2 = pltpu.async_copy(tz_hbm.at[idx_v.at[h]], rz_v, s2)
        cp1.wait()
        cp2.wait()
        base = wid * ROWS_PER_W + h * CHUNK
        pltpu.sync_copy(rp_v, gp_hbm.at[pl.ds(base, CHUNK)])
        pltpu.sync_copy(rz_v, gz_hbm.at[pl.ds(base, CHUNK)])


def kernel(feat_a_p, feat_a_z, feat_b_p, feat_b_z, iou, iou_threshold):
    thr = jnp.asarray(iou_threshold, jnp.float32).reshape(1)

    flat_idx, mask = pl.pallas_call(
        _tc_argmax_body,
        grid=(B,),
        in_specs=[
            pl.BlockSpec(memory_space=pltpu.SMEM),
            pl.BlockSpec((1, NA, NB), lambda b: (b, 0, 0)),
        ],
        out_specs=[
            pl.BlockSpec((1, NA, 1), lambda b: (b, 0, 0)),
            pl.BlockSpec((1, NA, 1), lambda b: (b, 0, 0)),
        ],
        out_shape=[
            jax.ShapeDtypeStruct((B, NA, 1), jnp.int32),
            jax.ShapeDtypeStruct((B, NA, 1), jnp.float32),
        ],
    )(thr, iou)

    idx_padded = jnp.concatenate(
        [flat_idx.reshape(B * NA), jnp.zeros((PAD - B * NA,), jnp.int32)]
    ).reshape(PAD // CHUNK, CHUNK)

    mesh = plsc.VectorSubcoreMesh(core_axis_name="c", subcore_axis_name="s")
    sc_gather = functools.partial(
        pl.kernel,
        out_type=[
            jax.ShapeDtypeStruct((PAD, D), jnp.float32),
            jax.ShapeDtypeStruct((PAD, D), jnp.float32),
        ],
        mesh=mesh,
        scratch_types=[
            pltpu.VMEM((2, CHUNK), jnp.int32),
            pltpu.VMEM((CHUNK, D), jnp.float32),
            pltpu.VMEM((CHUNK, D), jnp.float32),
            pltpu.SemaphoreType.DMA,
            pltpu.SemaphoreType.DMA,
        ],
    )(_sc_gather_body)
    gp_pad, gz_pad = sc_gather(
        feat_b_p.reshape(B * NB, D), feat_b_z.reshape(B * NB, D), idx_padded
    )

    sa, sb, cnt = pl.pallas_call(
        _tc_cosine_body,
        grid=(B,),
        in_specs=[
            pl.BlockSpec((1, NA, D), lambda b: (b, 0, 0)),
            pl.BlockSpec((1, NA, D), lambda b: (b, 0, 0)),
            pl.BlockSpec((NA, D), lambda b: (b, 0)),
            pl.BlockSpec((NA, D), lambda b: (b, 0)),
            pl.BlockSpec((1, NA, 1), lambda b: (b, 0, 0)),
        ],
        out_specs=[
            pl.BlockSpec((1, 8, 128), lambda b: (b, 0, 0)),
            pl.BlockSpec((1, 8, 128), lambda b: (b, 0, 0)),
            pl.BlockSpec((1, 8, 128), lambda b: (b, 0, 0)),
        ],
        out_shape=[
            jax.ShapeDtypeStruct((B, 8, 128), jnp.float32),
            jax.ShapeDtypeStruct((B, 8, 128), jnp.float32),
            jax.ShapeDtypeStruct((B, 8, 128), jnp.float32),
        ],
    )(feat_a_p, feat_a_z, gp_pad, gz_pad, mask)

    matched_box_num = cnt[:, 0, 0]
    denom = jnp.maximum(jnp.sum(matched_box_num), 1.0)
    loss = -(jnp.sum(sa[:, 0, 0]) + jnp.sum(sb[:, 0, 0])) / (2.0 * denom)
    return (loss, matched_box_num)
